# baseline (speedup 1.0000x reference)
import functools

import jax
import jax.numpy as jnp
from jax import lax
from jax.experimental import pallas as pl
from jax.experimental.pallas import tpu as pltpu
from jax.experimental.pallas import tpu_sc as plsc

NUM_EXPERTS = 8
TOP_K = 2
IN_DIM = 1024
HID_DIM = 2048
OUT_DIM = 1024
T_TOKENS = 4096
A_ASSIGN = T_TOKENS * TOP_K
TB = 512
TB_SHIFT = 9
NB = A_ASSIGN // TB + NUM_EXPERTS
P_ROWS = NB * TB

ROUTER_TM = 512
NW = 32
GCH = 96
GSUB = 16
GNST = GCH // GSUB
CCH = 16

_LANES = 16


def _router_body(x_ref, wg_ref, bg_ref, weights_ref, idxpad_ref, rankpad_ref,
                 wpad_ref, usage_ref, sched_ref):
    t = pl.program_id(0)
    scores = jax.lax.dot_general(
        x_ref[...], wg_ref[...], (((1,), (1,)), ((), ())),
        preferred_element_type=jnp.float32)
    scores = scores + bg_ref[...]
    tm = scores.shape[0]
    iota_e = jax.lax.broadcasted_iota(jnp.int32, (tm, _LANES), 1)
    scores = jnp.where(iota_e < NUM_EXPERTS, scores, -jnp.inf)
    m0 = jnp.max(scores, axis=1, keepdims=True)
    idx0 = jnp.min(jnp.where(scores == m0, iota_e, NUM_EXPERTS), axis=1,
                   keepdims=True)
    masked = jnp.where(iota_e == idx0, -jnp.inf, scores)
    m1 = jnp.max(masked, axis=1, keepdims=True)
    idx1 = jnp.min(jnp.where(masked == m1, iota_e, NUM_EXPERTS), axis=1,
                   keepdims=True)
    e1 = jnp.exp(m1 - m0)
    denom = 1.0 + e1
    w0 = 1.0 / denom
    w1 = e1 / denom
    onehot0 = (iota_e == idx0)
    onehot1 = (iota_e == idx1)
    weights_ref[...] = (jnp.where(onehot0, w0, 0.0)
                        + jnp.where(onehot1, w1, 0.0))
    idxpad_ref[...] = jnp.where(iota_e == 0, idx0, idx1)
    wpad_ref[...] = jnp.where(iota_e == 0, w0, w1)

    @pl.when(t == 0)
    def _():
        usage_ref[...] = jnp.zeros_like(usage_ref)

    carry = usage_ref[...]
    m_f = onehot0.astype(jnp.float32) + onehot1.astype(jnp.float32)
    iota_r = jax.lax.broadcasted_iota(jnp.int32, (tm, tm), 0)
    iota_c = jax.lax.broadcasted_iota(jnp.int32, (tm, tm), 1)
    tri = (iota_r > iota_c).astype(jnp.float32)
    prior = jax.lax.dot_general(
        tri, m_f, (((1,), (0,)), ((), ())),
        preferred_element_type=jnp.float32)
    cprior = carry.astype(jnp.float32) + prior
    rank0 = jnp.sum(jnp.where(onehot0, cprior, 0.0), axis=1,
                    keepdims=True).astype(jnp.int32)
    rank1 = jnp.sum(jnp.where(onehot1, cprior, 0.0), axis=1,
                    keepdims=True).astype(jnp.int32)
    rankpad_ref[...] = jnp.where(iota_e == 0, rank0, rank1)
    usage_t = (jnp.sum(onehot0.astype(jnp.int32), axis=0, keepdims=True)
               + jnp.sum(onehot1.astype(jnp.int32), axis=0, keepdims=True))
    usage_new = carry + usage_t
    usage_ref[...] = usage_new
    blocks = jax.lax.shift_right_logical(usage_new + (TB - 1), TB_SHIFT)
    iota_r16 = jax.lax.broadcasted_iota(jnp.int32, (_LANES, _LANES), 0)
    iota_c16 = jax.lax.broadcasted_iota(jnp.int32, (_LANES, _LANES), 1)
    tri16 = (iota_r16 <= iota_c16).astype(jnp.float32)
    incl = jax.lax.dot_general(
        blocks.astype(jnp.float32), tri16, (((1,), (0,)), ((), ())),
        preferred_element_type=jnp.float32).astype(jnp.int32)
    start = incl - blocks
    lane = jax.lax.broadcasted_iota(jnp.int32, (1, _LANES), 1)
    nb = jnp.sum(jnp.where(lane == NUM_EXPERTS - 1, incl, 0), axis=1,
                 keepdims=True)
    sched_ref[...] = jnp.where(lane < NUM_EXPERTS, start,
                               jnp.where(lane == NUM_EXPERTS, nb, 0))


def _sc_sort_body(ef_hbm, rf_hbm, wf_hbm, sched_hbm,
                  st_hbm, sw_hbm, p0_hbm, p1_hbm,
                  ef_v, rf_v, wf_v, sched_v, pado_v,
                  st_v, sw_v, p0_v, p1_v):
    cid = lax.axis_index("c")
    sid = lax.axis_index("s")

    @pl.when((cid == 0) & (sid == 0))
    def _():
        pltpu.sync_copy(ef_hbm, ef_v)
        pltpu.sync_copy(rf_hbm, rf_v)
        pltpu.sync_copy(wf_hbm, wf_v)
        pltpu.sync_copy(sched_hbm, sched_v)
        iota16 = lax.iota(jnp.int32, _LANES)
        pado_v[...] = jax.lax.shift_left(sched_v[...], TB_SHIFT)

        def init_body(i, carry):
            st_v[pl.ds(i * _LANES, _LANES)] = jnp.zeros((_LANES,), jnp.int32)
            return carry

        lax.fori_loop(0, P_ROWS // _LANES, init_body, 0)

        def body(i, carry):
            sl = pl.ds(i * _LANES, _LANES)
            e = ef_v[sl]
            r = rf_v[sl]
            w = wf_v[sl]
            po = plsc.load_gather(pado_v, [e])
            pos = po + r
            a = iota16 + i * _LANES
            tok = jax.lax.shift_right_logical(a, 1)
            even = (a & 1) == 0
            plsc.store_scatter(st_v, [pos], tok)
            plsc.store_scatter(sw_v, [pos], w)
            plsc.store_scatter(p0_v, [tok], pos, mask=even)
            plsc.store_scatter(p1_v, [tok], pos, mask=jnp.logical_not(even))
            return carry

        lax.fori_loop(0, A_ASSIGN // _LANES, body, 0)
        pltpu.sync_copy(st_v, st_hbm)
        pltpu.sync_copy(sw_v, sw_hbm)
        pltpu.sync_copy(p0_v, p0_hbm)
        pltpu.sync_copy(p1_v, p1_hbm)


def _sc_gather_body(x_hbm, st_hbm, sched_hbm, xs_hbm,
                    sched_v, *bufs, half):
    idx = list(bufs[:GNST])
    rows = list(bufs[GNST:2 * GNST])
    gsem = list(bufs[2 * GNST:3 * GNST])
    ssem = list(bufs[3 * GNST:4 * GNST])
    wid = lax.axis_index("s") * 2 + lax.axis_index("c")
    p_half = P_ROWS // 2
    rows_per_tile = p_half // NW
    nch = rows_per_tile // GCH
    pltpu.sync_copy(sched_hbm, sched_v)
    iota16 = lax.iota(jnp.int32, _LANES)
    arows = jnp.sum(
        jnp.where(iota16 == NUM_EXPERTS, sched_v[...], 0), axis=0) * TB
    for ch in range(nch):
        lbase = wid * rows_per_tile + ch * GCH
        gbase = half * p_half + lbase

        @pl.when(gbase < arows)
        def _(lbase=lbase, gbase=gbase, ch=ch):
            descs = []
            for s in range(GNST):
                pltpu.sync_copy(st_hbm.at[pl.ds(gbase + s * GSUB, GSUB)],
                                idx[s])
                if ch > 0:
                    pltpu.make_async_copy(
                        rows[s], xs_hbm.at[pl.ds(lbase - GCH + s * GSUB,
                                                 GSUB)], ssem[s]).wait()
                descs.append(
                    pltpu.async_copy(x_hbm.at[idx[s]], rows[s], gsem[s]))
            for s in range(GNST):
                descs[s].wait()
                pltpu.async_copy(rows[s],
                                 xs_hbm.at[pl.ds(lbase + s * GSUB, GSUB)],
                                 ssem[s])
    for s in range(GNST):
        @pl.when(half * p_half + wid * rows_per_tile < arows)
        def _(s=s):
            pltpu.make_async_copy(
                rows[s],
                xs_hbm.at[pl.ds(wid * rows_per_tile + s * GSUB, GSUB)],
                ssem[s]).wait()


def _ffn_compute(sched_ref, xs_ref, w1_ref, b1_ref, w2_ref, b2_ref, sw_ref,
                 ys_ref, goff):
    i = pl.program_id(0) + goff
    nb = sched_ref[NUM_EXPERTS]

    @pl.when(i < nb)
    def _():
        h = jax.lax.dot_general(
            xs_ref[...].astype(jnp.bfloat16), w1_ref[0].astype(jnp.bfloat16),
            (((1,), (1,)), ((), ())), preferred_element_type=jnp.float32)
        h = jnp.maximum(h + b1_ref[0], 0.0)
        y = jax.lax.dot_general(
            h.astype(jnp.bfloat16), w2_ref[0].astype(jnp.bfloat16),
            (((1,), (1,)), ((), ())), preferred_element_type=jnp.float32)
        ys_ref[...] = sw_ref[...] * (y + b2_ref[0])


def _ffn_body_lo(sched_ref, xs_ref, w1_ref, b1_ref, w2_ref, b2_ref, sw_ref,
                 ys_ref):
    _ffn_compute(sched_ref, xs_ref, w1_ref, b1_ref, w2_ref, b2_ref, sw_ref,
                 ys_ref, 0)


def _ffn_body_hi(sched_ref, xs_ref, w1_ref, b1_ref, w2_ref, b2_ref, sw_ref,
                 ysin_ref, ys_ref):
    del ysin_ref
    _ffn_compute(sched_ref, xs_ref, w1_ref, b1_ref, w2_ref, b2_ref, sw_ref,
                 ys_ref, NB // 2)


def _block_expert(i, sched_ref):
    e = jnp.int32(-1)
    for k in range(NUM_EXPERTS):
        e = e + jnp.where(i >= sched_ref[k], 1, 0).astype(jnp.int32)
    return e


def _sc_combine_body(ys_hbm, p0_hbm, p1_hbm, out_hbm,
                     i0a_v, i1a_v, i0b_v, i1b_v,
                     y0a_v, y1a_v, y0b_v, y1b_v,
                     g0a, g1a, g0b, g1b, ssa, ssb):
    wid = lax.axis_index("s") * 2 + lax.axis_index("c")
    tok_per_tile = T_TOKENS // NW
    nch = tok_per_tile // CCH
    i0 = [i0a_v, i0b_v]
    i1 = [i1a_v, i1b_v]
    y0 = [y0a_v, y0b_v]
    y1 = [y1a_v, y1b_v]
    g0s = [g0a, g0b]
    g1s = [g1a, g1b]
    ssem = [ssa, ssb]
    g0 = [None, None]
    g1 = [None, None]
    sd = [None, None]

    def _add_store(prv, pbase):
        g0[prv].wait()
        g1[prv].wait()
        for r in range(CCH):
            def colbody(j, carry, r=r, prv=prv):
                sl = pl.ds(j * _LANES, _LANES)
                y0[prv][r, sl] = y0[prv][r, sl] + y1[prv][r, sl]
                return carry

            lax.fori_loop(0, OUT_DIM // _LANES, colbody, 0)
        sd[prv] = pltpu.async_copy(y0[prv], out_hbm.at[pl.ds(pbase, CCH)],
                                   ssem[prv])

    for ch in range(nch):
        cur = ch & 1
        if sd[cur] is not None:
            sd[cur].wait()
        base = wid * tok_per_tile + ch * CCH
        pltpu.sync_copy(p0_hbm.at[pl.ds(base, CCH)], i0[cur])
        pltpu.sync_copy(p1_hbm.at[pl.ds(base, CCH)], i1[cur])
        g0[cur] = pltpu.async_copy(ys_hbm.at[i0[cur]], y0[cur], g0s[cur])
        g1[cur] = pltpu.async_copy(ys_hbm.at[i1[cur]], y1[cur], g1s[cur])
        if ch >= 1:
            _add_store(1 - cur, base - CCH)
    last = (nch - 1) & 1
    _add_store(last, wid * tok_per_tile + (nch - 1) * CCH)
    sd[0].wait()
    sd[1].wait()


@functools.cache
def _sc_kernels():
    mesh = plsc.VectorSubcoreMesh(core_axis_name="c", subcore_axis_name="s")
    params = pltpu.CompilerParams(needs_layout_passes=False)
    sc_sort = functools.partial(
        pl.kernel, _sc_sort_body, mesh=mesh,
        compiler_params=params,
        out_type=[
        jax.ShapeDtypeStruct((P_ROWS,), jnp.int32),
        jax.ShapeDtypeStruct((P_ROWS,), jnp.float32),
        jax.ShapeDtypeStruct((T_TOKENS,), jnp.int32),
        jax.ShapeDtypeStruct((T_TOKENS,), jnp.int32),
    ],
    scratch_types=[
        pltpu.VMEM((A_ASSIGN,), jnp.int32),
        pltpu.VMEM((A_ASSIGN,), jnp.int32),
        pltpu.VMEM((A_ASSIGN,), jnp.float32),
        pltpu.VMEM((_LANES,), jnp.int32),
        pltpu.VMEM((_LANES,), jnp.int32),
        pltpu.VMEM((P_ROWS,), jnp.int32),
        pltpu.VMEM((P_ROWS,), jnp.float32),
        pltpu.VMEM((T_TOKENS,), jnp.int32),
        pltpu.VMEM((T_TOKENS,), jnp.int32),
    ],
    )()

    def _mk_gather(half):
        return functools.partial(
            pl.kernel,
            functools.partial(_sc_gather_body, half=half), mesh=mesh,
            compiler_params=params,
            out_type=jax.ShapeDtypeStruct((P_ROWS // 2, IN_DIM),
                                          jnp.float32),
            scratch_types=(
                [pltpu.VMEM((_LANES,), jnp.int32)]
                + [pltpu.VMEM((GSUB,), jnp.int32) for _ in range(GNST)]
                + [pltpu.VMEM((GSUB, IN_DIM), jnp.float32)
                   for _ in range(GNST)]
                + [pltpu.SemaphoreType.DMA for _ in range(2 * GNST)]
            ),
        )()

    sc_gather = (_mk_gather(0), _mk_gather(1))

    sc_combine = functools.partial(
        pl.kernel, _sc_combine_body, mesh=mesh,
        compiler_params=params,
        out_type=jax.ShapeDtypeStruct((T_TOKENS, OUT_DIM), jnp.float32),
        scratch_types=(
            [pltpu.VMEM((CCH,), jnp.int32) for _ in range(4)]
            + [pltpu.VMEM((CCH, OUT_DIM), jnp.float32) for _ in range(4)]
            + [pltpu.SemaphoreType.DMA for _ in range(6)]
        ),
    )()
    return sc_sort, sc_gather, sc_combine


def _sc_sort(ef, rf, wf, sched):
    return _sc_kernels()[0](ef, rf, wf, sched)


def _sc_gather(xf, sorted_tok, sched, half):
    return _sc_kernels()[1][half](xf, sorted_tok, sched)


def _sc_combine(ys, pos0, pos1):
    return _sc_kernels()[2](ys, pos0, pos1)


def kernel(x, Wg, bg, W1, b1, W2, b2):
    B, S, D = x.shape
    T = B * S
    xf = x.reshape(T, D)

    wg16 = jnp.pad(Wg, ((0, _LANES - NUM_EXPERTS), (0, 0)))
    bg16 = jnp.pad(bg, (0, _LANES - NUM_EXPERTS))

    n_rt = T // ROUTER_TM
    weights16, idxpad, rankpad, wpad, _usage16, _sched2d = pl.pallas_call(
        _router_body,
        grid=(n_rt,),
        in_specs=[
            pl.BlockSpec((ROUTER_TM, D), lambda t: (t, 0)),
            pl.BlockSpec((_LANES, D), lambda t: (0, 0)),
            pl.BlockSpec((_LANES,), lambda t: (0,)),
        ],
        out_specs=[
            pl.BlockSpec((ROUTER_TM, _LANES), lambda t: (t, 0)),
            pl.BlockSpec((ROUTER_TM, _LANES), lambda t: (t, 0)),
            pl.BlockSpec((ROUTER_TM, _LANES), lambda t: (t, 0)),
            pl.BlockSpec((ROUTER_TM, _LANES), lambda t: (t, 0)),
            pl.BlockSpec((1, _LANES), lambda t: (0, 0)),
            pl.BlockSpec((1, _LANES), lambda t: (0, 0)),
        ],
        out_shape=[
            jax.ShapeDtypeStruct((T, _LANES), jnp.float32),
            jax.ShapeDtypeStruct((T, _LANES), jnp.int32),
            jax.ShapeDtypeStruct((T, _LANES), jnp.int32),
            jax.ShapeDtypeStruct((T, _LANES), jnp.float32),
            jax.ShapeDtypeStruct((1, _LANES), jnp.int32),
            jax.ShapeDtypeStruct((1, _LANES), jnp.int32),
        ],
    )(xf, wg16, bg16)

    weights = weights16[:, :NUM_EXPERTS]
    top_k_indices = idxpad[:, :TOP_K]
    expert_usage = _usage16.reshape(_LANES)[:NUM_EXPERTS]
    sched = _sched2d.reshape(_LANES)

    ef = top_k_indices.reshape(A_ASSIGN)
    rf = rankpad[:, :TOP_K].reshape(A_ASSIGN)
    wf = wpad[:, :TOP_K].reshape(A_ASSIGN)

    sorted_tok, sorted_w, pos0, pos1 = _sc_sort(ef, rf, wf, sched)

    b1r = b1.reshape(NUM_EXPERTS, 1, HID_DIM)
    b2r = b2.reshape(NUM_EXPERTS, 1, OUT_DIM)
    sw2d = sorted_w.reshape(P_ROWS, 1)
    NB2 = NB // 2

    def _ffn_specs(goff, with_ysin):
        in_specs = [
            pl.BlockSpec((TB, D), lambda i, s: (i, 0)),
            pl.BlockSpec((1, HID_DIM, D),
                         lambda i, s: (_block_expert(i + goff, s), 0, 0)),
            pl.BlockSpec((1, 1, HID_DIM),
                         lambda i, s: (_block_expert(i + goff, s), 0, 0)),
            pl.BlockSpec((1, OUT_DIM, HID_DIM),
                         lambda i, s: (_block_expert(i + goff, s), 0, 0)),
            pl.BlockSpec((1, 1, OUT_DIM),
                         lambda i, s: (_block_expert(i + goff, s), 0, 0)),
            pl.BlockSpec((TB, 1), lambda i, s: (i + goff, 0)),
        ]
        if with_ysin:
            in_specs.append(pl.BlockSpec((8, 128), lambda i, s: (0, 0)))
        return pltpu.PrefetchScalarGridSpec(
            num_scalar_prefetch=1,
            grid=(NB2,),
            in_specs=in_specs,
            out_specs=pl.BlockSpec((TB, OUT_DIM), lambda i, s: (i + goff, 0)),
        )

    xs1 = _sc_gather(xf, sorted_tok, sched, 0)
    ys_lo = pl.pallas_call(
        _ffn_body_lo,
        grid_spec=_ffn_specs(0, False),
        out_shape=jax.ShapeDtypeStruct((P_ROWS, OUT_DIM), jnp.float32),
    )(sched, xs1, W1, b1r, W2, b2r, sw2d)
    xs2 = _sc_gather(xf, sorted_tok, sched, 1)
    ys = pl.pallas_call(
        _ffn_body_hi,
        grid_spec=_ffn_specs(NB2, True),
        out_shape=jax.ShapeDtypeStruct((P_ROWS, OUT_DIM), jnp.float32),
        input_output_aliases={7: 0},
    )(sched, xs2, W1, b1r, W2, b2r, sw2d, ys_lo)

    out_flat = _sc_combine(ys, pos0, pos1)

    output = out_flat.reshape(B, S, OUT_DIM)
    return output, weights, expert_usage, top_k_indices


# speedup vs baseline: 1.0123x; 1.0123x over previous
import functools

import jax
import jax.numpy as jnp
from jax import lax
from jax.experimental import pallas as pl
from jax.experimental.pallas import tpu as pltpu
from jax.experimental.pallas import tpu_sc as plsc

NUM_EXPERTS = 8
TOP_K = 2
IN_DIM = 1024
HID_DIM = 2048
OUT_DIM = 1024
T_TOKENS = 4096
A_ASSIGN = T_TOKENS * TOP_K
TB = 512
TB_SHIFT = 9
NB = A_ASSIGN // TB + NUM_EXPERTS
P_ROWS = NB * TB

ROUTER_TM = 512
NW = 32
GCH = 96
GSUB = 32
GNST = GCH // GSUB
CCH = 16

_LANES = 16


def _router_body(x_ref, wg_ref, bg_ref, weights_ref, idxpad_ref, rankpad_ref,
                 wpad_ref, usage_ref, sched_ref):
    t = pl.program_id(0)
    scores = jax.lax.dot_general(
        x_ref[...], wg_ref[...], (((1,), (1,)), ((), ())),
        preferred_element_type=jnp.float32)
    scores = scores + bg_ref[...]
    tm = scores.shape[0]
    iota_e = jax.lax.broadcasted_iota(jnp.int32, (tm, _LANES), 1)
    scores = jnp.where(iota_e < NUM_EXPERTS, scores, -jnp.inf)
    m0 = jnp.max(scores, axis=1, keepdims=True)
    idx0 = jnp.min(jnp.where(scores == m0, iota_e, NUM_EXPERTS), axis=1,
                   keepdims=True)
    masked = jnp.where(iota_e == idx0, -jnp.inf, scores)
    m1 = jnp.max(masked, axis=1, keepdims=True)
    idx1 = jnp.min(jnp.where(masked == m1, iota_e, NUM_EXPERTS), axis=1,
                   keepdims=True)
    e1 = jnp.exp(m1 - m0)
    denom = 1.0 + e1
    w0 = 1.0 / denom
    w1 = e1 / denom
    onehot0 = (iota_e == idx0)
    onehot1 = (iota_e == idx1)
    weights_ref[...] = (jnp.where(onehot0, w0, 0.0)
                        + jnp.where(onehot1, w1, 0.0))
    idxpad_ref[...] = jnp.where(iota_e == 0, idx0, idx1)
    wpad_ref[...] = jnp.where(iota_e == 0, w0, w1)

    @pl.when(t == 0)
    def _():
        usage_ref[...] = jnp.zeros_like(usage_ref)

    carry = usage_ref[...]
    m_f = onehot0.astype(jnp.float32) + onehot1.astype(jnp.float32)
    iota_r = jax.lax.broadcasted_iota(jnp.int32, (tm, tm), 0)
    iota_c = jax.lax.broadcasted_iota(jnp.int32, (tm, tm), 1)
    tri = (iota_r > iota_c).astype(jnp.float32)
    prior = jax.lax.dot_general(
        tri, m_f, (((1,), (0,)), ((), ())),
        preferred_element_type=jnp.float32)
    cprior = carry.astype(jnp.float32) + prior
    rank0 = jnp.sum(jnp.where(onehot0, cprior, 0.0), axis=1,
                    keepdims=True).astype(jnp.int32)
    rank1 = jnp.sum(jnp.where(onehot1, cprior, 0.0), axis=1,
                    keepdims=True).astype(jnp.int32)
    rankpad_ref[...] = jnp.where(iota_e == 0, rank0, rank1)
    usage_t = (jnp.sum(onehot0.astype(jnp.int32), axis=0, keepdims=True)
               + jnp.sum(onehot1.astype(jnp.int32), axis=0, keepdims=True))
    usage_new = carry + usage_t
    usage_ref[...] = usage_new
    blocks = jax.lax.shift_right_logical(usage_new + (TB - 1), TB_SHIFT)
    iota_r16 = jax.lax.broadcasted_iota(jnp.int32, (_LANES, _LANES), 0)
    iota_c16 = jax.lax.broadcasted_iota(jnp.int32, (_LANES, _LANES), 1)
    tri16 = (iota_r16 <= iota_c16).astype(jnp.float32)
    incl = jax.lax.dot_general(
        blocks.astype(jnp.float32), tri16, (((1,), (0,)), ((), ())),
        preferred_element_type=jnp.float32).astype(jnp.int32)
    start = incl - blocks
    lane = jax.lax.broadcasted_iota(jnp.int32, (1, _LANES), 1)
    nb = jnp.sum(jnp.where(lane == NUM_EXPERTS - 1, incl, 0), axis=1,
                 keepdims=True)
    sched_ref[...] = jnp.where(lane < NUM_EXPERTS, start,
                               jnp.where(lane == NUM_EXPERTS, nb, 0))


def _sc_sort_body(ef_hbm, rf_hbm, wf_hbm, sched_hbm,
                  st_hbm, sw_hbm, p0_hbm, p1_hbm,
                  ef_v, rf_v, wf_v, sched_v, pado_v,
                  st_v, sw_v, p0_v, p1_v):
    cid = lax.axis_index("c")
    sid = lax.axis_index("s")

    @pl.when((cid == 0) & (sid == 0))
    def _():
        pltpu.sync_copy(ef_hbm, ef_v)
        pltpu.sync_copy(rf_hbm, rf_v)
        pltpu.sync_copy(wf_hbm, wf_v)
        pltpu.sync_copy(sched_hbm, sched_v)
        iota16 = lax.iota(jnp.int32, _LANES)
        pado_v[...] = jax.lax.shift_left(sched_v[...], TB_SHIFT)

        def init_body(i, carry):
            st_v[pl.ds(i * _LANES, _LANES)] = jnp.zeros((_LANES,), jnp.int32)
            return carry

        lax.fori_loop(0, P_ROWS // _LANES, init_body, 0)

        def body(i, carry):
            sl = pl.ds(i * _LANES, _LANES)
            e = ef_v[sl]
            r = rf_v[sl]
            w = wf_v[sl]
            po = plsc.load_gather(pado_v, [e])
            pos = po + r
            a = iota16 + i * _LANES
            tok = jax.lax.shift_right_logical(a, 1)
            even = (a & 1) == 0
            plsc.store_scatter(st_v, [pos], tok)
            plsc.store_scatter(sw_v, [pos], w)
            plsc.store_scatter(p0_v, [tok], pos, mask=even)
            plsc.store_scatter(p1_v, [tok], pos, mask=jnp.logical_not(even))
            return carry

        lax.fori_loop(0, A_ASSIGN // _LANES, body, 0)
        pltpu.sync_copy(st_v, st_hbm)
        pltpu.sync_copy(sw_v, sw_hbm)
        pltpu.sync_copy(p0_v, p0_hbm)
        pltpu.sync_copy(p1_v, p1_hbm)


def _sc_gather_body(x_hbm, st_hbm, sched_hbm, xs_hbm,
                    sched_v, *bufs, half):
    idx = list(bufs[:GNST])
    rows = list(bufs[GNST:2 * GNST])
    gsem = list(bufs[2 * GNST:3 * GNST])
    ssem = list(bufs[3 * GNST:4 * GNST])
    wid = lax.axis_index("s") * 2 + lax.axis_index("c")
    p_half = P_ROWS // 2
    rows_per_tile = p_half // NW
    nch = rows_per_tile // GCH
    pltpu.sync_copy(sched_hbm, sched_v)
    iota16 = lax.iota(jnp.int32, _LANES)
    arows = jnp.sum(
        jnp.where(iota16 == NUM_EXPERTS, sched_v[...], 0), axis=0) * TB
    for ch in range(nch):
        lbase = wid * rows_per_tile + ch * GCH
        gbase = half * p_half + lbase

        @pl.when(gbase < arows)
        def _(lbase=lbase, gbase=gbase, ch=ch):
            descs = []
            for s in range(GNST):
                pltpu.sync_copy(st_hbm.at[pl.ds(gbase + s * GSUB, GSUB)],
                                idx[s])
                if ch > 0:
                    pltpu.make_async_copy(
                        rows[s], xs_hbm.at[pl.ds(lbase - GCH + s * GSUB,
                                                 GSUB)], ssem[s]).wait()
                descs.append(
                    pltpu.async_copy(x_hbm.at[idx[s]], rows[s], gsem[s]))
            for s in range(GNST):
                descs[s].wait()
                pltpu.async_copy(rows[s],
                                 xs_hbm.at[pl.ds(lbase + s * GSUB, GSUB)],
                                 ssem[s])
    for s in range(GNST):
        @pl.when(half * p_half + wid * rows_per_tile < arows)
        def _(s=s):
            pltpu.make_async_copy(
                rows[s],
                xs_hbm.at[pl.ds(wid * rows_per_tile + s * GSUB, GSUB)],
                ssem[s]).wait()


def _ffn_compute(sched_ref, xs_ref, w1_ref, b1_ref, w2_ref, b2_ref, sw_ref,
                 ys_ref, goff):
    i = pl.program_id(0) + goff
    nb = sched_ref[NUM_EXPERTS]

    @pl.when(i < nb)
    def _():
        h = jax.lax.dot_general(
            xs_ref[...].astype(jnp.bfloat16), w1_ref[0].astype(jnp.bfloat16),
            (((1,), (1,)), ((), ())), preferred_element_type=jnp.float32)
        h = jnp.maximum(h + b1_ref[0], 0.0)
        y = jax.lax.dot_general(
            h.astype(jnp.bfloat16), w2_ref[0].astype(jnp.bfloat16),
            (((1,), (1,)), ((), ())), preferred_element_type=jnp.float32)
        ys_ref[...] = sw_ref[...] * (y + b2_ref[0])


def _ffn_body_lo(sched_ref, xs_ref, w1_ref, b1_ref, w2_ref, b2_ref, sw_ref,
                 ys_ref):
    _ffn_compute(sched_ref, xs_ref, w1_ref, b1_ref, w2_ref, b2_ref, sw_ref,
                 ys_ref, 0)


def _ffn_body_hi(sched_ref, xs_ref, w1_ref, b1_ref, w2_ref, b2_ref, sw_ref,
                 ysin_ref, ys_ref):
    del ysin_ref
    _ffn_compute(sched_ref, xs_ref, w1_ref, b1_ref, w2_ref, b2_ref, sw_ref,
                 ys_ref, NB // 2)


def _block_expert(i, sched_ref):
    e = jnp.int32(-1)
    for k in range(NUM_EXPERTS):
        e = e + jnp.where(i >= sched_ref[k], 1, 0).astype(jnp.int32)
    return e


def _sc_combine_body(ys_hbm, p0_hbm, p1_hbm, out_hbm,
                     i0a_v, i1a_v, i0b_v, i1b_v,
                     y0a_v, y1a_v, y0b_v, y1b_v,
                     g0a, g1a, g0b, g1b, ssa, ssb):
    wid = lax.axis_index("s") * 2 + lax.axis_index("c")
    tok_per_tile = T_TOKENS // NW
    nch = tok_per_tile // CCH
    i0 = [i0a_v, i0b_v]
    i1 = [i1a_v, i1b_v]
    y0 = [y0a_v, y0b_v]
    y1 = [y1a_v, y1b_v]
    g0s = [g0a, g0b]
    g1s = [g1a, g1b]
    ssem = [ssa, ssb]
    g0 = [None, None]
    g1 = [None, None]
    sd = [None, None]

    def _add_store(prv, pbase):
        g0[prv].wait()
        g1[prv].wait()
        for r in range(CCH):
            def colbody(j, carry, r=r, prv=prv):
                sl = pl.ds(j * _LANES, _LANES)
                y0[prv][r, sl] = y0[prv][r, sl] + y1[prv][r, sl]
                return carry

            lax.fori_loop(0, OUT_DIM // _LANES, colbody, 0)
        sd[prv] = pltpu.async_copy(y0[prv], out_hbm.at[pl.ds(pbase, CCH)],
                                   ssem[prv])

    for ch in range(nch):
        cur = ch & 1
        if sd[cur] is not None:
            sd[cur].wait()
        base = wid * tok_per_tile + ch * CCH
        pltpu.sync_copy(p0_hbm.at[pl.ds(base, CCH)], i0[cur])
        pltpu.sync_copy(p1_hbm.at[pl.ds(base, CCH)], i1[cur])
        g0[cur] = pltpu.async_copy(ys_hbm.at[i0[cur]], y0[cur], g0s[cur])
        g1[cur] = pltpu.async_copy(ys_hbm.at[i1[cur]], y1[cur], g1s[cur])
        if ch >= 1:
            _add_store(1 - cur, base - CCH)
    last = (nch - 1) & 1
    _add_store(last, wid * tok_per_tile + (nch - 1) * CCH)
    sd[0].wait()
    sd[1].wait()


@functools.cache
def _sc_kernels():
    mesh = plsc.VectorSubcoreMesh(core_axis_name="c", subcore_axis_name="s")
    params = pltpu.CompilerParams(needs_layout_passes=False)
    sc_sort = functools.partial(
        pl.kernel, _sc_sort_body, mesh=mesh,
        compiler_params=params,
        out_type=[
        jax.ShapeDtypeStruct((P_ROWS,), jnp.int32),
        jax.ShapeDtypeStruct((P_ROWS,), jnp.float32),
        jax.ShapeDtypeStruct((T_TOKENS,), jnp.int32),
        jax.ShapeDtypeStruct((T_TOKENS,), jnp.int32),
    ],
    scratch_types=[
        pltpu.VMEM((A_ASSIGN,), jnp.int32),
        pltpu.VMEM((A_ASSIGN,), jnp.int32),
        pltpu.VMEM((A_ASSIGN,), jnp.float32),
        pltpu.VMEM((_LANES,), jnp.int32),
        pltpu.VMEM((_LANES,), jnp.int32),
        pltpu.VMEM((P_ROWS,), jnp.int32),
        pltpu.VMEM((P_ROWS,), jnp.float32),
        pltpu.VMEM((T_TOKENS,), jnp.int32),
        pltpu.VMEM((T_TOKENS,), jnp.int32),
    ],
    )()

    def _mk_gather(half):
        return functools.partial(
            pl.kernel,
            functools.partial(_sc_gather_body, half=half), mesh=mesh,
            compiler_params=params,
            out_type=jax.ShapeDtypeStruct((P_ROWS // 2, IN_DIM),
                                          jnp.float32),
            scratch_types=(
                [pltpu.VMEM((_LANES,), jnp.int32)]
                + [pltpu.VMEM((GSUB,), jnp.int32) for _ in range(GNST)]
                + [pltpu.VMEM((GSUB, IN_DIM), jnp.float32)
                   for _ in range(GNST)]
                + [pltpu.SemaphoreType.DMA for _ in range(2 * GNST)]
            ),
        )()

    sc_gather = (_mk_gather(0), _mk_gather(1))

    sc_combine = functools.partial(
        pl.kernel, _sc_combine_body, mesh=mesh,
        compiler_params=params,
        out_type=jax.ShapeDtypeStruct((T_TOKENS, OUT_DIM), jnp.float32),
        scratch_types=(
            [pltpu.VMEM((CCH,), jnp.int32) for _ in range(4)]
            + [pltpu.VMEM((CCH, OUT_DIM), jnp.float32) for _ in range(4)]
            + [pltpu.SemaphoreType.DMA for _ in range(6)]
        ),
    )()
    return sc_sort, sc_gather, sc_combine


def _sc_sort(ef, rf, wf, sched):
    return _sc_kernels()[0](ef, rf, wf, sched)


def _sc_gather(xf, sorted_tok, sched, half):
    return _sc_kernels()[1][half](xf, sorted_tok, sched)


def _sc_combine(ys, pos0, pos1):
    return _sc_kernels()[2](ys, pos0, pos1)


def kernel(x, Wg, bg, W1, b1, W2, b2):
    B, S, D = x.shape
    T = B * S
    xf = x.reshape(T, D)

    wg16 = jnp.pad(Wg, ((0, _LANES - NUM_EXPERTS), (0, 0)))
    bg16 = jnp.pad(bg, (0, _LANES - NUM_EXPERTS))

    n_rt = T // ROUTER_TM
    weights16, idxpad, rankpad, wpad, _usage16, _sched2d = pl.pallas_call(
        _router_body,
        grid=(n_rt,),
        in_specs=[
            pl.BlockSpec((ROUTER_TM, D), lambda t: (t, 0)),
            pl.BlockSpec((_LANES, D), lambda t: (0, 0)),
            pl.BlockSpec((_LANES,), lambda t: (0,)),
        ],
        out_specs=[
            pl.BlockSpec((ROUTER_TM, _LANES), lambda t: (t, 0)),
            pl.BlockSpec((ROUTER_TM, _LANES), lambda t: (t, 0)),
            pl.BlockSpec((ROUTER_TM, _LANES), lambda t: (t, 0)),
            pl.BlockSpec((ROUTER_TM, _LANES), lambda t: (t, 0)),
            pl.BlockSpec((1, _LANES), lambda t: (0, 0)),
            pl.BlockSpec((1, _LANES), lambda t: (0, 0)),
        ],
        out_shape=[
            jax.ShapeDtypeStruct((T, _LANES), jnp.float32),
            jax.ShapeDtypeStruct((T, _LANES), jnp.int32),
            jax.ShapeDtypeStruct((T, _LANES), jnp.int32),
            jax.ShapeDtypeStruct((T, _LANES), jnp.float32),
            jax.ShapeDtypeStruct((1, _LANES), jnp.int32),
            jax.ShapeDtypeStruct((1, _LANES), jnp.int32),
        ],
    )(xf, wg16, bg16)

    weights = weights16[:, :NUM_EXPERTS]
    top_k_indices = idxpad[:, :TOP_K]
    expert_usage = _usage16.reshape(_LANES)[:NUM_EXPERTS]
    sched = _sched2d.reshape(_LANES)

    ef = top_k_indices.reshape(A_ASSIGN)
    rf = rankpad[:, :TOP_K].reshape(A_ASSIGN)
    wf = wpad[:, :TOP_K].reshape(A_ASSIGN)

    sorted_tok, sorted_w, pos0, pos1 = _sc_sort(ef, rf, wf, sched)

    b1r = b1.reshape(NUM_EXPERTS, 1, HID_DIM)
    b2r = b2.reshape(NUM_EXPERTS, 1, OUT_DIM)
    sw2d = sorted_w.reshape(P_ROWS, 1)
    NB2 = NB // 2

    def _ffn_specs(goff, with_ysin):
        in_specs = [
            pl.BlockSpec((TB, D), lambda i, s: (i, 0)),
            pl.BlockSpec((1, HID_DIM, D),
                         lambda i, s: (_block_expert(i + goff, s), 0, 0)),
            pl.BlockSpec((1, 1, HID_DIM),
                         lambda i, s: (_block_expert(i + goff, s), 0, 0)),
            pl.BlockSpec((1, OUT_DIM, HID_DIM),
                         lambda i, s: (_block_expert(i + goff, s), 0, 0)),
            pl.BlockSpec((1, 1, OUT_DIM),
                         lambda i, s: (_block_expert(i + goff, s), 0, 0)),
            pl.BlockSpec((TB, 1), lambda i, s: (i + goff, 0)),
        ]
        if with_ysin:
            in_specs.append(pl.BlockSpec((8, 128), lambda i, s: (0, 0)))
        return pltpu.PrefetchScalarGridSpec(
            num_scalar_prefetch=1,
            grid=(NB2,),
            in_specs=in_specs,
            out_specs=pl.BlockSpec((TB, OUT_DIM), lambda i, s: (i + goff, 0)),
        )

    xs1 = _sc_gather(xf, sorted_tok, sched, 0)
    ys_lo = pl.pallas_call(
        _ffn_body_lo,
        grid_spec=_ffn_specs(0, False),
        out_shape=jax.ShapeDtypeStruct((P_ROWS, OUT_DIM), jnp.float32),
    )(sched, xs1, W1, b1r, W2, b2r, sw2d)
    xs2 = _sc_gather(xf, sorted_tok, sched, 1)
    ys = pl.pallas_call(
        _ffn_body_hi,
        grid_spec=_ffn_specs(NB2, True),
        out_shape=jax.ShapeDtypeStruct((P_ROWS, OUT_DIM), jnp.float32),
        input_output_aliases={7: 0},
    )(sched, xs2, W1, b1r, W2, b2r, sw2d, ys_lo)

    out_flat = _sc_combine(ys, pos0, pos1)

    output = out_flat.reshape(B, S, OUT_DIM)
    return output, weights, expert_usage, top_k_indices
